# baseline (speedup 1.0000x reference)
import jax
import jax.numpy as jnp
from jax import lax
from jax.experimental import pallas as pl
from jax.experimental.pallas import tpu as pltpu
from jax.experimental.pallas import tpu_sc as plsc

_BATCH = 4096
_TABLE = 1001

_INFO = plsc.get_sparse_core_info()
_NS = _INFO.num_subcores
_L = _INFO.num_lanes
_USE_NC = 1
_NW = _USE_NC * _NS
_BPW = _BATCH // _NW
_HALF = _BPW // 2


def _gather_body(table_hbm, t_hbm, out_hbm, table_v, idx_v, out_v,
                 sem_t, sem_i, sem_o):
    wid = lax.axis_index("s") * _USE_NC + lax.axis_index("c")
    base = wid * _BPW
    cp_t = pltpu.async_copy(table_hbm, table_v, sem_t)
    cp_i = pltpu.async_copy(t_hbm.at[pl.ds(base, _BPW)], idx_v, sem_i)
    cp_i.wait()
    cp_t.wait()
    for j in range(_HALF // _L):
        idx = idx_v[pl.ds(j * _L, _L)]
        out_v[pl.ds(j * _L, _L)] = plsc.load_gather(table_v, [idx])
    cp_o = pltpu.async_copy(
        out_v.at[pl.ds(0, _HALF)], out_hbm.at[pl.ds(base, _HALF)], sem_o
    )
    for j in range(_HALF // _L, _BPW // _L):
        idx = idx_v[pl.ds(j * _L, _L)]
        out_v[pl.ds(j * _L, _L)] = plsc.load_gather(table_v, [idx])
    pltpu.sync_copy(
        out_v.at[pl.ds(_HALF, _HALF)], out_hbm.at[pl.ds(base + _HALF, _HALF)]
    )
    cp_o.wait()


@jax.jit
def _gather(table, t):
    mesh = plsc.VectorSubcoreMesh(
        core_axis_name="c", subcore_axis_name="s", num_cores=_USE_NC
    )
    return pl.kernel(
        _gather_body,
        mesh=mesh,
        out_type=jax.ShapeDtypeStruct((_BATCH,), jnp.float32),
        scratch_types=[
            pltpu.VMEM((_TABLE,), jnp.float32),
            pltpu.VMEM((_BPW,), jnp.int32),
            pltpu.VMEM((_BPW,), jnp.float32),
            pltpu.SemaphoreType.DMA,
            pltpu.SemaphoreType.DMA,
            pltpu.SemaphoreType.DMA,
        ],
        compiler_params=pltpu.CompilerParams(needs_layout_passes=False),
    )(table, t)


def kernel(t, alpha, alpha_bar):
    return _gather(alpha_bar, t.astype(jnp.int32))


# speedup vs baseline: 1.0012x; 1.0012x over previous
import jax
import jax.numpy as jnp
from jax import lax
from jax.experimental import pallas as pl
from jax.experimental.pallas import tpu as pltpu
from jax.experimental.pallas import tpu_sc as plsc

_BATCH = 4096
_TABLE = 1001

_INFO = plsc.get_sparse_core_info()
_NS = _INFO.num_subcores
_L = _INFO.num_lanes
_USE_NC = 1
_NW = _USE_NC * _NS
_BPW = _BATCH // _NW
_HALF = _BPW // 2


def _gather_body(table_hbm, t_hbm, out_hbm, table_v, idx_v, out_v, sem):
    wid = lax.axis_index("s") * _USE_NC + lax.axis_index("c")
    base = wid * _BPW
    cp_t = pltpu.async_copy(table_hbm, table_v, sem)
    cp_i = pltpu.async_copy(t_hbm.at[pl.ds(base, _BPW)], idx_v, sem)
    cp_i.wait()
    cp_t.wait()
    for j in range(_BPW // _L):
        idx = idx_v[pl.ds(j * _L, _L)]
        out_v[pl.ds(j * _L, _L)] = plsc.load_gather(table_v, [idx])
    pltpu.sync_copy(out_v, out_hbm.at[pl.ds(base, _BPW)])


@jax.jit
def _gather(table, t):
    mesh = plsc.VectorSubcoreMesh(
        core_axis_name="c", subcore_axis_name="s", num_cores=_USE_NC
    )
    return pl.kernel(
        _gather_body,
        mesh=mesh,
        out_type=jax.ShapeDtypeStruct((_BATCH,), jnp.float32),
        scratch_types=[
            pltpu.VMEM((_TABLE,), jnp.float32),
            pltpu.VMEM((_BPW,), jnp.int32),
            pltpu.VMEM((_BPW,), jnp.float32),
            pltpu.SemaphoreType.DMA,
        ],
        compiler_params=pltpu.CompilerParams(needs_layout_passes=False),
    )(table, t)


def kernel(t, alpha, alpha_bar):
    return _gather(alpha_bar, t.astype(jnp.int32))


# speedup vs baseline: 6.6248x; 6.6171x over previous
import jax
import jax.numpy as jnp
from jax.experimental import pallas as pl

_CH = 128
_NCH = 8
_ROWS = 32


def _take_body(tab_ref, t_ref, o_ref):
    t = t_ref[...]
    hi = t // _CH
    lo = t % _CH
    acc = jnp.zeros(t.shape, jnp.float32)
    for c in range(_NCH):
        chunk = jnp.broadcast_to(tab_ref[c].reshape(1, _CH), (_ROWS, _CH))
        g = jnp.take_along_axis(chunk, lo, axis=1, mode="promise_in_bounds")
        acc = jnp.where(hi == c, g, acc)
    o_ref[...] = acc


@jax.jit
def _take(tab, t):
    tab2 = jnp.zeros((_NCH * _CH,), jnp.float32).at[:1001].set(tab)
    return pl.pallas_call(
        _take_body, out_shape=jax.ShapeDtypeStruct((_ROWS, _CH), jnp.float32)
    )(tab2.reshape(_NCH, _CH), t.reshape(_ROWS, _CH))


def kernel(t, alpha, alpha_bar):
    return _take(alpha_bar, t.astype(jnp.int32)).reshape(4096)
